# Initial kernel scaffold; baseline (speedup 1.0000x reference)
import jax
import jax.numpy as jnp
from jax.experimental import pallas as pl


def kernel(x, W):
    raise NotImplementedError("write your pallas kernel here")


# speedup vs baseline: 2.2890x; 2.2890x over previous
import functools

import jax
import jax.numpy as jnp
from jax import lax
from jax.experimental import pallas as pl


def _logits_kernel(x_ref, w_ref, out_ref):
    xb = x_ref[0]
    wp = w_ref[...]
    r = jax.lax.dot_general(
        xb, wp, (((1,), (0,)), ((), ())),
        preferred_element_type=jnp.float32)
    out_ref[0, 0, :, :] = r[:, 0:1]


def _mask_kernel(k, logits_ref, mask_ref):
    logits = logits_ref[...]
    s = lax.bitcast_convert_type(logits, jnp.int32)

    count_pos = jnp.sum((s >= 0).astype(jnp.int32), axis=1, keepdims=True)
    use_pos = count_pos >= k
    kk = jnp.where(use_pos, k, k - count_pos)

    key = jnp.where(use_pos,
                    jnp.where(s >= 0, s, -1),
                    jnp.where(s < 0, ~s, -1))

    def body(i, v):
        bit = 30 - i
        cand = v | (jnp.int32(1) << bit)
        c = jnp.sum((key >= cand).astype(jnp.int32), axis=1, keepdims=True)
        return jnp.where(c >= kk, cand, v)

    v0 = jnp.zeros_like(count_pos)
    v = lax.fori_loop(0, 31, body, v0)

    pos_arm = jnp.where(s >= v, jnp.int32(1), jnp.int32(0))
    neg_arm = jnp.where((s >= 0) | (~s >= v), jnp.int32(1), jnp.int32(0))
    mask_ref[...] = jnp.where(use_pos, pos_arm, neg_arm)


def kernel(x, W):
    B, T, D = x.shape
    k = max(1, int(T * 0.5))
    Tt = 1024

    Wp = jnp.zeros((D, 128), jnp.float32).at[:, 0].set(W[0])
    logits = pl.pallas_call(
        _logits_kernel,
        grid=(B, T // Tt),
        in_specs=[
            pl.BlockSpec((1, Tt, D), lambda b, t: (b, t, 0)),
            pl.BlockSpec((D, 128), lambda b, t: (0, 0)),
        ],
        out_specs=pl.BlockSpec((1, 1, Tt, 1), lambda b, t: (b, t, 0, 0)),
        out_shape=jax.ShapeDtypeStruct((B, T // Tt, Tt, 1), jnp.float32),
    )(x, Wp).reshape(B, T)

    mask_i32 = pl.pallas_call(
        functools.partial(_mask_kernel, k),
        in_specs=[pl.BlockSpec((B, T), lambda: (0, 0))],
        out_specs=pl.BlockSpec((B, T), lambda: (0, 0)),
        out_shape=jax.ShapeDtypeStruct((B, T), jnp.int32),
    )(logits)

    return (mask_i32.astype(jnp.bool_), logits)


# speedup vs baseline: 2.5852x; 1.1294x over previous
import functools

import jax
import jax.numpy as jnp
from jax import lax
from jax.experimental import pallas as pl


def _logits_kernel(x_ref, w_ref, out_ref):
    xb = x_ref[0]
    wp = w_ref[...]
    r = jax.lax.dot_general(
        xb, wp, (((1,), (0,)), ((), ())),
        preferred_element_type=jnp.float32)
    out_ref[0, 0, :, :] = r[:, 0:1]


def _mask_kernel(k, logits_ref, mask_ref):
    logits = logits_ref[...]
    s = lax.bitcast_convert_type(logits, jnp.int32)

    count_pos = jnp.sum((s >= 0).astype(jnp.int32), axis=1, keepdims=True)
    use_pos = count_pos >= k
    kk = jnp.where(use_pos, k, k - count_pos)

    key = jnp.where(use_pos,
                    jnp.where(s >= 0, s, -1),
                    jnp.where(s < 0, ~s, -1))

    def body(i, v):
        bit = 30 - i
        cand = v | (jnp.int32(1) << bit)
        c = jnp.sum((key >= cand).astype(jnp.int32), axis=1, keepdims=True)
        return jnp.where(c >= kk, cand, v)

    v0 = jnp.zeros_like(count_pos)
    v = lax.fori_loop(0, 31, body, v0)

    pos_arm = jnp.where(s >= v, jnp.int32(1), jnp.int32(0))
    neg_arm = jnp.where((s >= 0) | (~s >= v), jnp.int32(1), jnp.int32(0))
    mask_ref[...] = jnp.where(use_pos, pos_arm, neg_arm)


def kernel(x, W):
    B, T, D = x.shape
    k = max(1, int(T * 0.5))
    Tt = 8192

    Wp = jnp.zeros((D, 128), jnp.float32).at[:, 0].set(W[0])
    logits = pl.pallas_call(
        _logits_kernel,
        grid=(B, T // Tt),
        in_specs=[
            pl.BlockSpec((1, Tt, D), lambda b, t: (b, t, 0)),
            pl.BlockSpec((D, 128), lambda b, t: (0, 0)),
        ],
        out_specs=pl.BlockSpec((1, 1, Tt, 1), lambda b, t: (b, t, 0, 0)),
        out_shape=jax.ShapeDtypeStruct((B, T // Tt, Tt, 1), jnp.float32),
    )(x, Wp).reshape(B, T)

    mask_i32 = pl.pallas_call(
        functools.partial(_mask_kernel, k),
        in_specs=[pl.BlockSpec((B, T), lambda: (0, 0))],
        out_specs=pl.BlockSpec((B, T), lambda: (0, 0)),
        out_shape=jax.ShapeDtypeStruct((B, T), jnp.int32),
    )(logits)

    return (mask_i32.astype(jnp.bool_), logits)
